# baseline (speedup 1.0000x reference)
import functools

import jax
import jax.numpy as jnp
from jax import lax
from jax.experimental import pallas as pl
from jax.experimental.pallas import tpu as pltpu
from jax.experimental.pallas import tpu_sc as plsc

NC, NS = 2, 16
NW = NC * NS
CHUNK = 128


def _repack_body(d, d_pad, x_ref, out_ref):
    x = x_ref[...]
    g = 128 // d_pad
    h = x.shape[1] // g
    zpad = jnp.zeros((d_pad - d, h), jnp.float32)
    parts = []
    for q in range(g):
        parts.append(x[:, q * h:(q + 1) * h])
        parts.append(zpad)
    xs = jnp.concatenate(parts, axis=0)
    out_ref[...] = jnp.transpose(xs)


def _repack(table_t, d_pad, block_c):
    d, n = table_t.shape
    nb = pl.cdiv(n, block_c)
    g = 128 // d_pad
    out_rows = nb * (block_c // g)
    return pl.pallas_call(
        functools.partial(_repack_body, d, d_pad),
        grid=(nb,),
        in_specs=[pl.BlockSpec((d, block_c), lambda i: (0, i))],
        out_specs=pl.BlockSpec((block_c // g, 128), lambda i: (i, 0)),
        out_shape=jax.ShapeDtypeStruct((out_rows, 128), jnp.float32),
        compiler_params=pltpu.CompilerParams(
            dimension_semantics=("parallel",)),
    )(table_t)


def _perm_rows(i, block_c, d_pad):
    g = 128 // d_pad
    if g == 1:
        return i
    h = block_c // g
    c = i // block_c
    r = i % block_c
    q = r // h
    m = r % h
    return (h * c + m) * g + q


def _sc_gather_body(dims, segs, n_out, *refs):
    n_tab = len(dims)
    tables = refs[:n_tab]
    idxs = refs[n_tab:2 * n_tab]
    outs = refs[2 * n_tab:2 * n_tab + n_out]
    idx_vs = refs[2 * n_tab + n_out:3 * n_tab + n_out]
    rows = refs[3 * n_tab + n_out:4 * n_tab + n_out]
    sems = refs[4 * n_tab + n_out:5 * n_tab + n_out]

    b_per_w = idx_vs[0].shape[0]
    n_chunks = b_per_w // CHUNK
    wid = lax.axis_index("s") * NC + lax.axis_index("c")
    base = wid * b_per_w

    def fire(t, cch, buf):
        return pltpu.async_copy(
            tables[t].at[idx_vs[t].at[pl.ds(cch * CHUNK, CHUNK)]],
            rows[t].at[buf], sems[t])

    for t in range(n_tab):
        pltpu.sync_copy(idxs[t].at[pl.ds(base, b_per_w)], idx_vs[t])
    cps = {}
    for t in range(n_tab):
        cps[(t, 0)] = fire(t, 0, 0)
        if n_chunks > 1:
            cps[(t, 1)] = fire(t, 1, 1)
    for cch in range(n_chunks):
        for t in range(n_tab):
            cps.pop((t, cch)).wait()
            for (o, col, src, w) in segs[t]:
                src_ref = rows[t].at[cch % 2]
                if (src, w) != (0, dims[t]):
                    src_ref = src_ref.at[:, pl.ds(src, w)]
                pltpu.sync_copy(
                    src_ref,
                    outs[o].at[pl.ds(base + cch * CHUNK, CHUNK),
                               pl.ds(col, w)])
            if cch + 2 < n_chunks:
                cps[(t, cch + 2)] = fire(t, cch + 2, cch % 2)


def _sc_gather(tables, idxs, segs, n_out, tc_tiling=False):
    B = idxs[0].shape[0]
    dims = tuple(t.shape[1] for t in tables)
    b_per_w = B // NW
    mesh = plsc.VectorSubcoreMesh(core_axis_name="c", subcore_axis_name="s")

    out_type = tuple(
        jax.ShapeDtypeStruct((B, 128), jnp.float32) for _ in range(n_out))
    scratch = (
        [pltpu.VMEM((b_per_w,), jnp.int32) for _ in tables]
        + [pltpu.VMEM((2, CHUNK, d), jnp.float32) for d in dims]
        + [pltpu.SemaphoreType.DMA for _ in tables]
    )
    kern = pl.kernel(
        functools.partial(_sc_gather_body, dims, segs, n_out),
        out_type=out_type,
        mesh=mesh,
        scratch_types=scratch,
        compiler_params=pltpu.CompilerParams(
            use_tc_tiling_on_sc=tc_tiling),
    )
    return kern(*tables, *idxs)


def _mlp_body(g0, g1, g2, g3, w0, w1, w2_, w3, b1, w2, b2, wo, bo, out):
    acc = jnp.dot(g0[...], w0[...], preferred_element_type=jnp.float32)
    acc += jnp.dot(g1[...], w1[...], preferred_element_type=jnp.float32)
    acc += jnp.dot(g2[...], w2_[...], preferred_element_type=jnp.float32)
    acc += jnp.dot(g3[...], w3[...], preferred_element_type=jnp.float32)
    h1 = jnp.maximum(acc + b1[...], 0.0)
    h2 = jnp.maximum(
        jnp.dot(h1, w2[...], preferred_element_type=jnp.float32)
        + b2[...], 0.0)
    out[...] = jnp.dot(h2, wo[...], preferred_element_type=jnp.float32) \
        + bo[...]


def _tc_mlp(gs, w1s, b1, W2, b2, Wo, bo, block_b=2048):
    B = gs[0].shape[0]
    nb = B // block_b

    def full(a):
        return pl.BlockSpec(a.shape, lambda i: (0,) * a.ndim)

    return pl.pallas_call(
        _mlp_body,
        grid=(nb,),
        in_specs=[pl.BlockSpec((block_b, 128), lambda i: (i, 0))
                  for _ in gs]
                 + [full(w) for w in w1s]
                 + [full(b1), full(W2), full(b2), full(Wo), full(bo)],
        out_specs=pl.BlockSpec((block_b, 1), lambda i: (i, 0)),
        out_shape=jax.ShapeDtypeStruct((B, 1), jnp.float32),
        compiler_params=pltpu.CompilerParams(
            dimension_semantics=("parallel",)),
    )(*gs, *w1s, b1, W2, b2, Wo, bo)


def kernel(x, cell_emb, assay_emb, p25_emb, p250_emb, p5k_emb,
           W1, b1, W2, b2, Wo, bo):
    x = x.astype(jnp.int32)
    B = x.shape[0]
    i_cell, i_assay, i25, i250, i5k = (x[:, j] for j in range(5))

    ga0, ga1 = _sc_gather((assay_emb,), (i_assay,),
                          [[(0, 0, 0, 128), (1, 0, 128, 128)]], 2,
                          tc_tiling=True)

    BC = 8192
    p25_r = _repack(p25_emb.T, 32, BC)
    p250_r = _repack(p250_emb.T, 64, BC)
    p5k_r = _repack(p5k_emb.T, 128, BC)

    p25_v = p25_r.reshape(-1, 32)
    p250_v = p250_r.reshape(-1, 64)

    gb0, gb1 = _sc_gather(
        (p25_v, cell_emb, p250_v, p5k_r),
        (_perm_rows(i25, BC, 32), i_cell, _perm_rows(i250, BC, 64), i5k),
        [[(0, 0, 0, 32)], [(0, 32, 0, 32)], [(0, 64, 0, 64)],
         [(1, 0, 0, 128)]], 2)

    w1c = W1[0:32]
    w1a = W1[32:288]
    w1p = W1[288:313]
    w1q = W1[313:353]
    w1r = W1[353:398]
    wb0 = jnp.concatenate([
        jnp.pad(w1p, ((0, 7), (0, 0))),
        w1c,
        jnp.pad(w1q, ((0, 24), (0, 0))),
    ], axis=0)
    wb1 = jnp.pad(w1r, ((0, 83), (0, 0)))

    return _tc_mlp((ga0, ga1, gb0, gb1),
                   (w1a[0:128], w1a[128:256], wb0, wb1),
                   b1.reshape(1, -1), W2, b2.reshape(1, -1),
                   Wo, bo.reshape(1, 1))


# speedup vs baseline: 1.2600x; 1.2600x over previous
import functools

import jax
import jax.numpy as jnp
from jax import lax
from jax.experimental import pallas as pl
from jax.experimental.pallas import tpu as pltpu
from jax.experimental.pallas import tpu_sc as plsc

NC, NS = 2, 16
NW = NC * NS
CHUNK = 128


def _repack_body(d, d_pad, x_ref, out_ref):
    x = x_ref[...]
    g = 128 // d_pad
    h = x.shape[1] // g
    zpad = jnp.zeros((d_pad - d, h), jnp.float32)
    parts = []
    for q in range(g):
        parts.append(x[:, q * h:(q + 1) * h])
        parts.append(zpad)
    xs = jnp.concatenate(parts, axis=0)
    out_ref[...] = jnp.transpose(xs)


def _repack(table_t, d_pad, block_c):
    d, n = table_t.shape
    nb = pl.cdiv(n, block_c)
    g = 128 // d_pad
    out_rows = nb * (block_c // g)
    return pl.pallas_call(
        functools.partial(_repack_body, d, d_pad),
        grid=(nb,),
        in_specs=[pl.BlockSpec((d, block_c), lambda i: (0, i))],
        out_specs=pl.BlockSpec((block_c // g, 128), lambda i: (i, 0)),
        out_shape=jax.ShapeDtypeStruct((out_rows, 128), jnp.float32),
        compiler_params=pltpu.CompilerParams(
            dimension_semantics=("parallel",)),
    )(table_t)


def _perm_rows(i, block_c, d_pad):
    g = 128 // d_pad
    if g == 1:
        return i
    h = block_c // g
    c = i // block_c
    r = i % block_c
    q = r // h
    m = r % h
    return (h * c + m) * g + q


def _sc_gather_body(dims, segs, n_out, *refs):
    n_tab = len(dims)
    tables = refs[:n_tab]
    idxs = refs[n_tab:2 * n_tab]
    outs = refs[2 * n_tab:2 * n_tab + n_out]
    idx_vs = refs[2 * n_tab + n_out:3 * n_tab + n_out]
    rows = refs[3 * n_tab + n_out:4 * n_tab + n_out]
    sems = refs[4 * n_tab + n_out:5 * n_tab + n_out]

    b_per_w = idx_vs[0].shape[0]
    n_chunks = b_per_w // CHUNK
    wid = lax.axis_index("s") * NC + lax.axis_index("c")
    base = wid * b_per_w

    nbufs = [r.shape[0] for r in rows]

    def fire(t, cch):
        return pltpu.async_copy(
            tables[t].at[idx_vs[t].at[pl.ds(cch * CHUNK, CHUNK)]],
            rows[t].at[cch % nbufs[t]], sems[t])

    for t in range(n_tab):
        pltpu.sync_copy(idxs[t].at[pl.ds(base, b_per_w)], idx_vs[t])
    cps = {}
    for t in range(n_tab):
        for cch in range(min(nbufs[t], n_chunks)):
            cps[(t, cch)] = fire(t, cch)
    for cch in range(n_chunks):
        for t in range(n_tab):
            cps.pop((t, cch)).wait()
            for (o, col, src, w) in segs[t]:
                src_ref = rows[t].at[cch % nbufs[t]]
                if (src, w) != (0, dims[t]):
                    src_ref = src_ref.at[:, pl.ds(src, w)]
                pltpu.sync_copy(
                    src_ref,
                    outs[o].at[pl.ds(base + cch * CHUNK, CHUNK),
                               pl.ds(col, w)])
            if cch + nbufs[t] < n_chunks:
                cps[(t, cch + nbufs[t])] = fire(t, cch + nbufs[t])


def _sc_gather(tables, idxs, segs, n_out, tc_tiling=False):
    B = idxs[0].shape[0]
    dims = tuple(t.shape[1] for t in tables)
    b_per_w = B // NW
    mesh = plsc.VectorSubcoreMesh(core_axis_name="c", subcore_axis_name="s")

    out_type = tuple(
        jax.ShapeDtypeStruct((B, 128), jnp.float32) for _ in range(n_out))
    scratch = (
        [pltpu.VMEM((b_per_w,), jnp.int32) for _ in tables]
        + [pltpu.VMEM((1 if d >= 256 else 2, CHUNK, d), jnp.float32)
           for d in dims]
        + [pltpu.SemaphoreType.DMA for _ in tables]
    )
    kern = pl.kernel(
        functools.partial(_sc_gather_body, dims, segs, n_out),
        out_type=out_type,
        mesh=mesh,
        scratch_types=scratch,
        compiler_params=pltpu.CompilerParams(
            use_tc_tiling_on_sc=tc_tiling),
    )
    return kern(*tables, *idxs)


def _mlp_body(g0, g1, g2, g3, w0, w1, w2_, w3, b1, w2, b2, wo, bo, out):
    acc = jnp.dot(g0[...], w0[...], preferred_element_type=jnp.float32)
    acc += jnp.dot(g1[...], w1[...], preferred_element_type=jnp.float32)
    acc += jnp.dot(g2[...], w2_[...], preferred_element_type=jnp.float32)
    acc += jnp.dot(g3[...], w3[...], preferred_element_type=jnp.float32)
    h1 = jnp.maximum(acc + b1[...], 0.0)
    h2 = jnp.maximum(
        jnp.dot(h1, w2[...], preferred_element_type=jnp.float32)
        + b2[...], 0.0)
    out[...] = jnp.dot(h2, wo[...], preferred_element_type=jnp.float32) \
        + bo[...]


def _tc_mlp(gs, w1s, b1, W2, b2, Wo, bo, block_b=4096):
    B = gs[0].shape[0]
    nb = B // block_b

    def full(a):
        return pl.BlockSpec(a.shape, lambda i: (0,) * a.ndim)

    return pl.pallas_call(
        _mlp_body,
        grid=(nb,),
        in_specs=[pl.BlockSpec((block_b, 128), lambda i: (i, 0))
                  for _ in gs]
                 + [full(w) for w in w1s]
                 + [full(b1), full(W2), full(b2), full(Wo), full(bo)],
        out_specs=pl.BlockSpec((block_b, 1), lambda i: (i, 0)),
        out_shape=jax.ShapeDtypeStruct((B, 1), jnp.float32),
        compiler_params=pltpu.CompilerParams(
            dimension_semantics=("parallel",)),
    )(*gs, *w1s, b1, W2, b2, Wo, bo)


def kernel(x, cell_emb, assay_emb, p25_emb, p250_emb, p5k_emb,
           W1, b1, W2, b2, Wo, bo):
    x = x.astype(jnp.int32)
    B = x.shape[0]
    i_cell, i_assay, i25, i250, i5k = (x[:, j] for j in range(5))

    BC = 16384
    p25_r = _repack(p25_emb.T, 32, BC)
    p250_r = _repack(p250_emb.T, 64, BC)
    p5k_r = _repack(p5k_emb.T, 128, BC)

    p25_v = p25_r.reshape(-1, 32)
    p250_v = p250_r.reshape(-1, 64)

    ga0, ga1, gb0, gb1 = _sc_gather(
        (assay_emb, p25_v, cell_emb, p250_v, p5k_r),
        (i_assay, _perm_rows(i25, BC, 32), i_cell,
         _perm_rows(i250, BC, 64), i5k),
        [[(0, 0, 0, 128), (1, 0, 128, 128)],
         [(2, 0, 0, 32)], [(2, 32, 0, 32)], [(2, 64, 0, 64)],
         [(3, 0, 0, 128)]], 4)

    w1c = W1[0:32]
    w1a = W1[32:288]
    w1p = W1[288:313]
    w1q = W1[313:353]
    w1r = W1[353:398]
    wb0 = jnp.concatenate([
        jnp.pad(w1p, ((0, 7), (0, 0))),
        w1c,
        jnp.pad(w1q, ((0, 24), (0, 0))),
    ], axis=0)
    wb1 = jnp.pad(w1r, ((0, 83), (0, 0)))

    return _tc_mlp((ga0, ga1, gb0, gb1),
                   (w1a[0:128], w1a[128:256], wb0, wb1),
                   b1.reshape(1, -1), W2, b2.reshape(1, -1),
                   Wo, bo.reshape(1, 1))
